# baseline (speedup 1.0000x reference)
import functools

import jax
import jax.numpy as jnp
from jax import lax
from jax.experimental import pallas as pl
from jax.experimental.pallas import tpu as pltpu
from jax.experimental.pallas import tpu_sc as plsc

_N = 10000
_E = 320000
_F = 128
_G = 128
_C = 10

_NC = 2
_NS = 16
_NW = _NC * _NS
_N_PAD = 10240
_RPT = _N_PAD // _NS
_EPW = _E // _NW
_K = 80
_CH = _EPW // _K
_ZR = 80


def _agg_body(h_hbm, src_hbm, dst_hbm, out_hbm,
              src_all, rows0, rows1, dst0, dst1,
              acc_sh, gsem0, gsem1, ssem0, ssem1, dsem0, dsem1, zsem):
    c = lax.axis_index("c")
    s = lax.axis_index("s")
    wid = s * _NC + c
    ebase = wid * _EPW
    rows_b = (rows0, rows1)
    dst_b = (dst0, dst1)
    gsem_b = (gsem0, gsem1)
    ssem_b = (ssem0, ssem1)
    dsem_b = (dsem0, dsem1)

    def start_gather(ch, b):
        pltpu.async_copy(dst_hbm.at[pl.ds(ebase + ch * _K, _K)],
                         dst_b[b], dsem_b[b])
        pltpu.async_copy(h_hbm.at[src_all.at[pl.ds(ch * _K, _K)]],
                         rows_b[b], gsem_b[b])

    def wait_gather(ch, b):
        pltpu.make_async_copy(dst_hbm.at[pl.ds(ebase + ch * _K, _K)],
                              dst_b[b], dsem_b[b]).wait()
        pltpu.make_async_copy(h_hbm.at[src_all.at[pl.ds(ch * _K, _K)]],
                              rows_b[b], gsem_b[b]).wait()

    def start_scatter(ch, b):
        pltpu.async_copy(rows_b[b], acc_sh.at[dst_b[b]], ssem_b[b],
                         add=True)

    def wait_scatter(b):
        pltpu.make_async_copy(rows_b[b], acc_sh.at[dst_b[b]],
                              ssem_b[b]).wait()

    pltpu.sync_copy(src_hbm.at[pl.ds(ebase, _EPW)], src_all)

    start_gather(0, 0)

    def zrow(r, carry):
        for cc in range(_F // 16):
            rows1[r, pl.ds(cc * 16, 16)] = jnp.zeros((16,), jnp.float32)
        return carry

    lax.fori_loop(0, _ZR, zrow, 0)
    nz = _RPT // _ZR
    for k in range(nz - 1):
        pltpu.async_copy(rows1, acc_sh.at[pl.ds(s * _RPT + k * _ZR, _ZR)],
                         zsem)
    pltpu.async_copy(rows1, acc_sh.at[pl.ds(s * _RPT + (nz - 1) * _ZR, _ZR)],
                     ssem1)
    for k in range(nz - 1):
        pltpu.make_async_copy(rows1,
                              acc_sh.at[pl.ds(s * _RPT + k * _ZR, _ZR)],
                              zsem).wait()
    plsc.subcore_barrier()

    def pair(g, carry):
        for j in range(2):
            ch = g * 2 + j
            wait_scatter(1 - j)
            start_gather(ch + 1, 1 - j)
            wait_gather(ch, j)
            start_scatter(ch, j)
        return carry

    lax.fori_loop(0, (_CH - 1) // 2, pair, 0)
    wait_gather(_CH - 1, 0)
    start_scatter(_CH - 1, 0)
    wait_scatter(0)
    wait_scatter(1)
    plsc.subcore_barrier()

    pltpu.sync_copy(
        acc_sh.at[pl.ds(s * _RPT, _RPT)],
        out_hbm.at[pl.ds(c * _N_PAD + s * _RPT, _RPT)],
    )


@functools.cache
def _make_agg():
    mesh = plsc.VectorSubcoreMesh(
        core_axis_name="c", subcore_axis_name="s",
        num_cores=_NC, num_subcores=_NS)
    return pl.kernel(
        _agg_body,
        mesh=mesh,
        out_type=jax.ShapeDtypeStruct((_NC * _N_PAD, _F), jnp.float32),
        scratch_types=[
            pltpu.VMEM((_EPW,), jnp.int32),
            pltpu.VMEM((_K, _F), jnp.float32),
            pltpu.VMEM((_K, _F), jnp.float32),
            pltpu.VMEM((_K,), jnp.int32),
            pltpu.VMEM((_K,), jnp.int32),
            pltpu.VMEM_SHARED((_N_PAD, _F), jnp.float32),
        ] + [pltpu.SemaphoreType.DMA] * 7,
    )


def _agg(h, src, dst):
    return _make_agg()(h, src, dst)


_BLK = 2000


def _mlp_body(h_ref, p_ref, w1_ref, b1_ref, w2_ref, b2_ref, o_ref):
    t = h_ref[...] + p_ref[0] + p_ref[1]
    a = jnp.maximum(
        jnp.dot(t, w1_ref[...], preferred_element_type=jnp.float32) + b1_ref[...], 0.0)
    o_ref[...] = jnp.maximum(
        jnp.dot(a, w2_ref[...], preferred_element_type=jnp.float32) + b2_ref[...], 0.0)


def _mlp_pool_head_body(h_ref, p_ref, w1_ref, b1_ref, w2_ref, b2_ref, bat_ref,
                        f1w_ref, f1b_ref, f2w_ref, f2b_ref, o_ref, pool_ref):
    i = pl.program_id(0)
    t = h_ref[...] + p_ref[0] + p_ref[1]
    a = jnp.maximum(
        jnp.dot(t, w1_ref[...], preferred_element_type=jnp.float32) + b1_ref[...], 0.0)
    o = jnp.maximum(
        jnp.dot(a, w2_ref[...], preferred_element_type=jnp.float32) + b2_ref[...], 0.0)
    onehot = (bat_ref[...] == lax.broadcasted_iota(jnp.int32, (_BLK, _G), 1)
              ).astype(jnp.float32)
    contrib = lax.dot_general(onehot, o, (((0,), (0,)), ((), ())),
                              preferred_element_type=jnp.float32)

    @pl.when(i == 0)
    def _():
        pool_ref[...] = jnp.zeros_like(pool_ref)

    pool_ref[...] += contrib

    @pl.when(i == _N // _BLK - 1)
    def _():
        ah = jnp.maximum(
            jnp.dot(pool_ref[...], f1w_ref[...],
                    preferred_element_type=jnp.float32) + f1b_ref[...], 0.0)
        logits = jnp.dot(ah, f2w_ref[...],
                         preferred_element_type=jnp.float32) + f2b_ref[...]
        m = jnp.max(logits, axis=-1, keepdims=True)
        lse = jnp.log(jnp.sum(jnp.exp(logits - m), axis=-1, keepdims=True)) + m
        o_ref[...] = logits - lse


def _run_mlp(h, part3, w1, b1, w2, b2):
    grid = (_N // _BLK,)
    return pl.pallas_call(
        _mlp_body,
        grid=grid,
        in_specs=[
            pl.BlockSpec((_BLK, _F), lambda i: (i, 0)),
            pl.BlockSpec((_NC, _BLK, _F), lambda i: (0, i, 0)),
            pl.BlockSpec((_F, _F), lambda i: (0, 0)),
            pl.BlockSpec((1, _F), lambda i: (0, 0)),
            pl.BlockSpec((_F, _F), lambda i: (0, 0)),
            pl.BlockSpec((1, _F), lambda i: (0, 0)),
        ],
        out_specs=pl.BlockSpec((_BLK, _F), lambda i: (i, 0)),
        out_shape=jax.ShapeDtypeStruct((_N, _F), jnp.float32),
    )(h, part3, w1, b1.reshape(1, _F), w2, b2.reshape(1, _F))


def _run_mlp_pool_head(h, part3, w1, b1, w2, b2, bat2,
                       fc1_w, fc1_b, fc2_wp, fc2_bp):
    grid = (_N // _BLK,)
    return pl.pallas_call(
        _mlp_pool_head_body,
        grid=grid,
        in_specs=[
            pl.BlockSpec((_BLK, _F), lambda i: (i, 0)),
            pl.BlockSpec((_NC, _BLK, _F), lambda i: (0, i, 0)),
            pl.BlockSpec((_F, _F), lambda i: (0, 0)),
            pl.BlockSpec((1, _F), lambda i: (0, 0)),
            pl.BlockSpec((_F, _F), lambda i: (0, 0)),
            pl.BlockSpec((1, _F), lambda i: (0, 0)),
            pl.BlockSpec((_BLK, 1), lambda i: (i, 0)),
            pl.BlockSpec((_F, _F), lambda i: (0, 0)),
            pl.BlockSpec((1, _F), lambda i: (0, 0)),
            pl.BlockSpec((_F, _F), lambda i: (0, 0)),
            pl.BlockSpec((1, _F), lambda i: (0, 0)),
        ],
        out_specs=pl.BlockSpec((_G, _F), lambda i: (0, 0)),
        out_shape=jax.ShapeDtypeStruct((_G, _F), jnp.float32),
        scratch_shapes=[pltpu.VMEM((_G, _F), jnp.float32)],
    )(h, part3, w1, b1.reshape(1, _F), w2, b2.reshape(1, _F), bat2,
      fc1_w, fc1_b.reshape(1, _F), fc2_wp, fc2_bp.reshape(1, _F))


def kernel(x, edge_index, batch,
           gin0_W1, gin0_b1, gin0_W2, gin0_b2,
           gin1_W1, gin1_b1, gin1_W2, gin1_b2,
           gin2_W1, gin2_b1, gin2_W2, gin2_b2,
           fc1_W, fc1_b, fc2_W, fc2_b):
    src = edge_index[0]
    dst = edge_index[1]
    bat2 = batch.reshape(_N, 1)
    fc2_wp = jnp.pad(fc2_W, ((0, 0), (0, _F - _C)))
    fc2_bp = jnp.pad(fc2_b, (0, _F - _C), constant_values=-1e30)

    layers = [(gin0_W1, gin0_b1, gin0_W2, gin0_b2),
              (gin1_W1, gin1_b1, gin1_W2, gin1_b2),
              (gin2_W1, gin2_b1, gin2_W2, gin2_b2)]

    h = x
    for li, (w1, b1, w2, b2) in enumerate(layers):
        part = _agg(h, src, dst)
        part3 = part.reshape(_NC, _N_PAD, _F)
        if li < 2:
            h = _run_mlp(h, part3, w1, b1, w2, b2)
        else:
            out = _run_mlp_pool_head(h, part3, w1, b1, w2, b2, bat2,
                                     fc1_W, fc1_b, fc2_wp, fc2_bp)

    return out[:, :_C]


# speedup vs baseline: 1.1994x; 1.1994x over previous
import functools

import jax
import jax.numpy as jnp
from jax import lax
from jax.experimental import pallas as pl
from jax.experimental.pallas import tpu as pltpu
from jax.experimental.pallas import tpu_sc as plsc

_N = 10000
_E = 320000
_F = 128
_G = 128
_C = 10

_NC = 2
_NS = 16
_NW = _NC * _NS
_N_PAD = 10240
_RPT = _N_PAD // _NS
_EPW = _E // _NW
_K = 80
_CH = _EPW // _K
_ZR = 80


def _agg_body(h_hbm, src_hbm, dst_hbm, out_hbm,
              src_all, rows0, rows1, rows2, dst0, dst1, dst2,
              acc_sh, gsem0, gsem1, gsem2, ssem0, ssem1, ssem2,
              dsem0, dsem1, dsem2, zsem):
    c = lax.axis_index("c")
    s = lax.axis_index("s")
    wid = s * _NC + c
    ebase = wid * _EPW
    rows_b = (rows0, rows1, rows2)
    dst_b = (dst0, dst1, dst2)
    gsem_b = (gsem0, gsem1, gsem2)
    ssem_b = (ssem0, ssem1, ssem2)
    dsem_b = (dsem0, dsem1, dsem2)

    def start_gather(ch, b):
        pltpu.async_copy(dst_hbm.at[pl.ds(ebase + ch * _K, _K)],
                         dst_b[b], dsem_b[b])
        pltpu.async_copy(h_hbm.at[src_all.at[pl.ds(ch * _K, _K)]],
                         rows_b[b], gsem_b[b])

    def wait_gather(ch, b):
        pltpu.make_async_copy(dst_hbm.at[pl.ds(ebase + ch * _K, _K)],
                              dst_b[b], dsem_b[b]).wait()
        pltpu.make_async_copy(h_hbm.at[src_all.at[pl.ds(ch * _K, _K)]],
                              rows_b[b], gsem_b[b]).wait()

    def start_scatter(ch, b):
        pltpu.async_copy(rows_b[b], acc_sh.at[dst_b[b]], ssem_b[b],
                         add=True)

    def wait_scatter(b):
        pltpu.make_async_copy(rows_b[b], acc_sh.at[dst_b[b]],
                              ssem_b[b]).wait()

    pltpu.sync_copy(src_hbm.at[pl.ds(ebase, _EPW)], src_all)

    start_gather(0, 0)
    start_gather(1, 1)

    def zrow(r, carry):
        for cc in range(_F // 16):
            rows2[r, pl.ds(cc * 16, 16)] = jnp.zeros((16,), jnp.float32)
        return carry

    lax.fori_loop(0, _ZR, zrow, 0)
    nz = _RPT // _ZR
    for k in range(nz - 1):
        pltpu.async_copy(rows2, acc_sh.at[pl.ds(s * _RPT + k * _ZR, _ZR)],
                         zsem)
    pltpu.async_copy(rows2, acc_sh.at[pl.ds(s * _RPT + (nz - 1) * _ZR, _ZR)],
                     ssem2)
    for k in range(nz - 1):
        pltpu.make_async_copy(rows2,
                              acc_sh.at[pl.ds(s * _RPT + k * _ZR, _ZR)],
                              zsem).wait()
    plsc.subcore_barrier()

    def trip(g, carry):
        for j in range(3):
            ch = g * 3 + j
            bn = (j + 2) % 3
            wait_scatter(bn)
            start_gather(ch + 2, bn)
            wait_gather(ch, j)
            start_scatter(ch, j)
        return carry

    lax.fori_loop(0, (_CH - 2) // 3, trip, 0)
    for ch in (_CH - 2, _CH - 1):
        j = ch % 3
        wait_gather(ch, j)
        start_scatter(ch, j)
    for j in range(3):
        wait_scatter(j)
    plsc.subcore_barrier()

    pltpu.sync_copy(
        acc_sh.at[pl.ds(s * _RPT, _RPT)],
        out_hbm.at[pl.ds(c * _N_PAD + s * _RPT, _RPT)],
    )


@functools.cache
def _make_agg():
    mesh = plsc.VectorSubcoreMesh(
        core_axis_name="c", subcore_axis_name="s",
        num_cores=_NC, num_subcores=_NS)
    return pl.kernel(
        _agg_body,
        mesh=mesh,
        out_type=jax.ShapeDtypeStruct((_NC * _N_PAD, _F), jnp.float32),
        scratch_types=[
            pltpu.VMEM((_EPW,), jnp.int32),
            pltpu.VMEM((_K, _F), jnp.float32),
            pltpu.VMEM((_K, _F), jnp.float32),
            pltpu.VMEM((_K, _F), jnp.float32),
            pltpu.VMEM((_K,), jnp.int32),
            pltpu.VMEM((_K,), jnp.int32),
            pltpu.VMEM((_K,), jnp.int32),
            pltpu.VMEM_SHARED((_N_PAD, _F), jnp.float32),
        ] + [pltpu.SemaphoreType.DMA] * 10,
    )


def _agg(h, src, dst):
    return _make_agg()(h, src, dst)


_BLK = 2000


def _mlp_body(h_ref, p_ref, w1_ref, b1_ref, w2_ref, b2_ref, o_ref):
    t = h_ref[...] + p_ref[0] + p_ref[1]
    a = jnp.maximum(
        jnp.dot(t, w1_ref[...], preferred_element_type=jnp.float32) + b1_ref[...], 0.0)
    o_ref[...] = jnp.maximum(
        jnp.dot(a, w2_ref[...], preferred_element_type=jnp.float32) + b2_ref[...], 0.0)


def _mlp_pool_head_body(h_ref, p_ref, w1_ref, b1_ref, w2_ref, b2_ref, bat_ref,
                        f1w_ref, f1b_ref, f2w_ref, f2b_ref, o_ref, pool_ref):
    i = pl.program_id(0)
    t = h_ref[...] + p_ref[0] + p_ref[1]
    a = jnp.maximum(
        jnp.dot(t, w1_ref[...], preferred_element_type=jnp.float32) + b1_ref[...], 0.0)
    o = jnp.maximum(
        jnp.dot(a, w2_ref[...], preferred_element_type=jnp.float32) + b2_ref[...], 0.0)
    onehot = (bat_ref[...] == lax.broadcasted_iota(jnp.int32, (_BLK, _G), 1)
              ).astype(jnp.float32)
    contrib = lax.dot_general(onehot, o, (((0,), (0,)), ((), ())),
                              preferred_element_type=jnp.float32)

    @pl.when(i == 0)
    def _():
        pool_ref[...] = jnp.zeros_like(pool_ref)

    pool_ref[...] += contrib

    @pl.when(i == _N // _BLK - 1)
    def _():
        ah = jnp.maximum(
            jnp.dot(pool_ref[...], f1w_ref[...],
                    preferred_element_type=jnp.float32) + f1b_ref[...], 0.0)
        logits = jnp.dot(ah, f2w_ref[...],
                         preferred_element_type=jnp.float32) + f2b_ref[...]
        m = jnp.max(logits, axis=-1, keepdims=True)
        lse = jnp.log(jnp.sum(jnp.exp(logits - m), axis=-1, keepdims=True)) + m
        o_ref[...] = logits - lse


def _run_mlp(h, part3, w1, b1, w2, b2):
    grid = (_N // _BLK,)
    return pl.pallas_call(
        _mlp_body,
        grid=grid,
        in_specs=[
            pl.BlockSpec((_BLK, _F), lambda i: (i, 0)),
            pl.BlockSpec((_NC, _BLK, _F), lambda i: (0, i, 0)),
            pl.BlockSpec((_F, _F), lambda i: (0, 0)),
            pl.BlockSpec((1, _F), lambda i: (0, 0)),
            pl.BlockSpec((_F, _F), lambda i: (0, 0)),
            pl.BlockSpec((1, _F), lambda i: (0, 0)),
        ],
        out_specs=pl.BlockSpec((_BLK, _F), lambda i: (i, 0)),
        out_shape=jax.ShapeDtypeStruct((_N, _F), jnp.float32),
    )(h, part3, w1, b1.reshape(1, _F), w2, b2.reshape(1, _F))


def _run_mlp_pool_head(h, part3, w1, b1, w2, b2, bat2,
                       fc1_w, fc1_b, fc2_wp, fc2_bp):
    grid = (_N // _BLK,)
    return pl.pallas_call(
        _mlp_pool_head_body,
        grid=grid,
        in_specs=[
            pl.BlockSpec((_BLK, _F), lambda i: (i, 0)),
            pl.BlockSpec((_NC, _BLK, _F), lambda i: (0, i, 0)),
            pl.BlockSpec((_F, _F), lambda i: (0, 0)),
            pl.BlockSpec((1, _F), lambda i: (0, 0)),
            pl.BlockSpec((_F, _F), lambda i: (0, 0)),
            pl.BlockSpec((1, _F), lambda i: (0, 0)),
            pl.BlockSpec((_BLK, 1), lambda i: (i, 0)),
            pl.BlockSpec((_F, _F), lambda i: (0, 0)),
            pl.BlockSpec((1, _F), lambda i: (0, 0)),
            pl.BlockSpec((_F, _F), lambda i: (0, 0)),
            pl.BlockSpec((1, _F), lambda i: (0, 0)),
        ],
        out_specs=pl.BlockSpec((_G, _F), lambda i: (0, 0)),
        out_shape=jax.ShapeDtypeStruct((_G, _F), jnp.float32),
        scratch_shapes=[pltpu.VMEM((_G, _F), jnp.float32)],
    )(h, part3, w1, b1.reshape(1, _F), w2, b2.reshape(1, _F), bat2,
      fc1_w, fc1_b.reshape(1, _F), fc2_wp, fc2_bp.reshape(1, _F))


def kernel(x, edge_index, batch,
           gin0_W1, gin0_b1, gin0_W2, gin0_b2,
           gin1_W1, gin1_b1, gin1_W2, gin1_b2,
           gin2_W1, gin2_b1, gin2_W2, gin2_b2,
           fc1_W, fc1_b, fc2_W, fc2_b):
    src = edge_index[0]
    dst = edge_index[1]
    bat2 = batch.reshape(_N, 1)
    fc2_wp = jnp.pad(fc2_W, ((0, 0), (0, _F - _C)))
    fc2_bp = jnp.pad(fc2_b, (0, _F - _C), constant_values=-1e30)

    layers = [(gin0_W1, gin0_b1, gin0_W2, gin0_b2),
              (gin1_W1, gin1_b1, gin1_W2, gin1_b2),
              (gin2_W1, gin2_b1, gin2_W2, gin2_b2)]

    h = x
    for li, (w1, b1, w2, b2) in enumerate(layers):
        part = _agg(h, src, dst)
        part3 = part.reshape(_NC, _N_PAD, _F)
        if li < 2:
            h = _run_mlp(h, part3, w1, b1, w2, b2)
        else:
            out = _run_mlp_pool_head(h, part3, w1, b1, w2, b2, bat2,
                                     fc1_W, fc1_b, fc2_wp, fc2_bp)

    return out[:, :_C]
